# baseline (speedup 1.0000x reference)
import functools

import jax
import jax.numpy as jnp
from jax import lax
from jax.experimental import pallas as pl
from jax.experimental.pallas import tpu as pltpu
from jax.experimental.pallas import tpu_sc as plsc

NC = 2
NS = 16
NW = NC * NS
LANES = 16

BATCH = 4
SEQ_LEN = 4096
D_MODEL = 2048
ROWS_PER_W = SEQ_LEN // NW
TILE_R = 16
N_TILES = ROWS_PER_W // TILE_R
VECS_PER_ROW = D_MODEL // LANES


def _sc_body(x_hbm, pos_hbm, out_hbm, pos_v, x_v):
    wid = lax.axis_index("s") * NC + lax.axis_index("c")
    s0 = wid * ROWS_PER_W

    def tile_body(t, _):
        r0 = s0 + t * TILE_R
        pltpu.sync_copy(pos_hbm.at[pl.ds(r0, TILE_R)], pos_v)

        def batch_body(b, _):
            row = b * SEQ_LEN + r0
            pltpu.sync_copy(x_hbm.at[pl.ds(row, TILE_R)], x_v)

            def add_body(i, _):
                r = i // VECS_PER_ROW
                j = (i % VECS_PER_ROW) * LANES
                x_v[r, pl.ds(j, LANES)] = (
                    x_v[r, pl.ds(j, LANES)] + pos_v[r, pl.ds(j, LANES)]
                )
                return 0

            lax.fori_loop(0, TILE_R * VECS_PER_ROW, add_body, 0)
            pltpu.sync_copy(x_v, out_hbm.at[pl.ds(row, TILE_R)])
            return 0

        lax.fori_loop(0, BATCH, batch_body, 0)
        return 0

    lax.fori_loop(0, N_TILES, tile_body, 0)


@functools.partial(jax.jit, static_argnames=())
def _sc_add(x2, pos_table):
    k = pl.kernel(
        _sc_body,
        out_type=jax.ShapeDtypeStruct((BATCH * SEQ_LEN, D_MODEL), jnp.float32),
        mesh=plsc.VectorSubcoreMesh(core_axis_name="c", subcore_axis_name="s"),
        scratch_types=[
            pltpu.VMEM((TILE_R, D_MODEL), jnp.float32),
            pltpu.VMEM((TILE_R, D_MODEL), jnp.float32),
        ],
    )
    return k(x2, pos_table)


def kernel(x, pos_table):
    batch, seq_len, d_model = x.shape
    x2 = x.reshape(batch * seq_len, d_model)
    out = _sc_add(x2, pos_table)
    return out.reshape(batch, seq_len, d_model)


# speedup vs baseline: 1.9821x; 1.9821x over previous
import functools

import jax
import jax.numpy as jnp
from jax import lax
from jax.experimental import pallas as pl
from jax.experimental.pallas import tpu as pltpu
from jax.experimental.pallas import tpu_sc as plsc

NC = 2
NS = 16
NW = NC * NS
LANES = 16

BATCH = 4
SEQ_LEN = 4096
D_MODEL = 2048
ROWS_PER_W = SEQ_LEN // NW
TILE_R = 16
N_TILES = ROWS_PER_W // TILE_R
VECS_PER_ROW = D_MODEL // LANES


def _sc_body(x_hbm, pos_hbm, out_hbm, pos_v, x_v):
    wid = lax.axis_index("s") * NC + lax.axis_index("c")
    s0 = wid * ROWS_PER_W

    def tile_body(t, _):
        r0 = s0 + t * TILE_R
        pltpu.sync_copy(pos_hbm.at[pl.ds(r0, TILE_R)], pos_v)

        def batch_body(b, _):
            row = b * SEQ_LEN + r0
            pltpu.sync_copy(x_hbm.at[pl.ds(row, TILE_R)], x_v)

            def row_body(r, _):
                @plsc.parallel_loop(0, D_MODEL, step=LANES, unroll=8)
                def _(j):
                    x_v[r, pl.ds(j, LANES)] = (
                        x_v[r, pl.ds(j, LANES)] + pos_v[r, pl.ds(j, LANES)]
                    )

                return 0

            lax.fori_loop(0, TILE_R, row_body, 0)
            pltpu.sync_copy(x_v, out_hbm.at[pl.ds(row, TILE_R)])
            return 0

        lax.fori_loop(0, BATCH, batch_body, 0)
        return 0

    lax.fori_loop(0, N_TILES, tile_body, 0)


@functools.partial(jax.jit, static_argnames=())
def _sc_add(x2, pos_table):
    k = pl.kernel(
        _sc_body,
        out_type=jax.ShapeDtypeStruct((BATCH * SEQ_LEN, D_MODEL), jnp.float32),
        mesh=plsc.VectorSubcoreMesh(core_axis_name="c", subcore_axis_name="s"),
        scratch_types=[
            pltpu.VMEM((TILE_R, D_MODEL), jnp.float32),
            pltpu.VMEM((TILE_R, D_MODEL), jnp.float32),
        ],
    )
    return k(x2, pos_table)


def kernel(x, pos_table):
    batch, seq_len, d_model = x.shape
    x2 = x.reshape(batch * seq_len, d_model)
    out = _sc_add(x2, pos_table)
    return out.reshape(batch, seq_len, d_model)


# speedup vs baseline: 2.6650x; 1.3445x over previous
import functools

import jax
import jax.numpy as jnp
from jax import lax
from jax.experimental import pallas as pl
from jax.experimental.pallas import tpu as pltpu
from jax.experimental.pallas import tpu_sc as plsc

NC = 2
NS = 16
NW = NC * NS
LANES = 16

BATCH = 4
SEQ_LEN = 4096
D_MODEL = 2048
ROWS_PER_W = SEQ_LEN // NW
TILE_R = 16
N_TILES = ROWS_PER_W // TILE_R
VECS_PER_ROW = D_MODEL // LANES


N_CHUNKS = N_TILES * BATCH


def _sc_body(x_hbm, pos_hbm, out_hbm, pos_v, x_v0, x_v1,
             ld0, ld1, st0, st1):
    wid = lax.axis_index("s") * NC + lax.axis_index("c")
    s0 = wid * ROWS_PER_W

    def x_row(k):
        t = k // BATCH
        b = k - t * BATCH
        return b * SEQ_LEN + s0 + t * TILE_R

    def start_load(k, buf, sem):
        pltpu.async_copy(x_hbm.at[pl.ds(x_row(k), TILE_R)], buf, sem)

    def start_store(k, buf, sem):
        pltpu.async_copy(buf, out_hbm.at[pl.ds(x_row(k), TILE_R)], sem)

    def wait(src, dst, sem):
        pltpu.make_async_copy(src, dst, sem).wait()

    def compute(k, buf):
        t = k // BATCH

        @pl.when(k - t * BATCH == 0)
        def _():
            pltpu.sync_copy(pos_hbm.at[pl.ds(s0 + t * TILE_R, TILE_R)], pos_v)

        def row_body(r, _):
            @plsc.parallel_loop(0, D_MODEL, step=LANES, unroll=8)
            def _(j):
                buf[r, pl.ds(j, LANES)] = (
                    buf[r, pl.ds(j, LANES)] + pos_v[r, pl.ds(j, LANES)]
                )

            return 0

        lax.fori_loop(0, TILE_R, row_body, 0)

    start_load(0, x_v0, ld0)
    start_load(1, x_v1, ld1)

    def pair_body(p, _):
        k0 = p * 2
        wait(x_hbm.at[pl.ds(x_row(k0), TILE_R)], x_v0, ld0)
        compute(k0, x_v0)
        start_store(k0, x_v0, st0)

        wait(x_hbm.at[pl.ds(x_row(k0 + 1), TILE_R)], x_v1, ld1)
        compute(k0 + 1, x_v1)
        start_store(k0 + 1, x_v1, st1)

        @pl.when(k0 + 2 < N_CHUNKS)
        def _():
            wait(x_v0, out_hbm.at[pl.ds(x_row(k0), TILE_R)], st0)
            start_load(k0 + 2, x_v0, ld0)

        @pl.when(k0 + 3 < N_CHUNKS)
        def _():
            wait(x_v1, out_hbm.at[pl.ds(x_row(k0 + 1), TILE_R)], st1)
            start_load(k0 + 3, x_v1, ld1)

        return 0

    lax.fori_loop(0, N_CHUNKS // 2, pair_body, 0)

    last = N_CHUNKS - 2
    wait(x_v0, out_hbm.at[pl.ds(x_row(last), TILE_R)], st0)
    wait(x_v1, out_hbm.at[pl.ds(x_row(last + 1), TILE_R)], st1)


@functools.partial(jax.jit, static_argnames=())
def _sc_add(x2, pos_table):
    k = pl.kernel(
        _sc_body,
        out_type=jax.ShapeDtypeStruct((BATCH * SEQ_LEN, D_MODEL), jnp.float32),
        mesh=plsc.VectorSubcoreMesh(core_axis_name="c", subcore_axis_name="s"),
        scratch_types=[
            pltpu.VMEM((TILE_R, D_MODEL), jnp.float32),
            pltpu.VMEM((TILE_R, D_MODEL), jnp.float32),
            pltpu.VMEM((TILE_R, D_MODEL), jnp.float32),
            pltpu.SemaphoreType.DMA,
            pltpu.SemaphoreType.DMA,
            pltpu.SemaphoreType.DMA,
            pltpu.SemaphoreType.DMA,
        ],
    )
    return k(x2, pos_table)


def kernel(x, pos_table):
    batch, seq_len, d_model = x.shape
    x2 = x.reshape(batch * seq_len, d_model)
    out = _sc_add(x2, pos_table)
    return out.reshape(batch, seq_len, d_model)
